# baseline (speedup 1.0000x reference)
import functools

import jax
import jax.numpy as jnp
from jax import lax
from jax.experimental import pallas as pl
from jax.experimental.pallas import tpu as pltpu
from jax.experimental.pallas import tpu_sc as plsc

VOCAB = 1000000
EMBED_DIM = 64
BATCH = 4096
SEQ = 200

_NC = 2
_NS = 16
_NW = _NC * _NS

_TOTAL = BATCH * SEQ
_PER_W = _TOTAL // _NW
_GATHER = 128
_K = 5
_CHUNK = _K * _GATHER
_NCHUNK = _PER_W // _CHUNK


def _embed_gather(idx_hbm, table_hbm, out_hbm, idx_v, rows_v, sem):
    wid = lax.axis_index("s") * _NC + lax.axis_index("c")
    base_w = wid * _PER_W

    def chunk_body(i, carry):
        base = base_w + i * _CHUNK
        pltpu.sync_copy(idx_hbm.at[pl.ds(base, _CHUNK)], idx_v)
        dmas = []
        for j in range(_K):
            dmas.append(pltpu.async_copy(
                table_hbm.at[idx_v.at[pl.ds(j * _GATHER, _GATHER)]],
                rows_v.at[pl.ds(j * _GATHER, _GATHER)],
                sem,
            ))
        for d in dmas:
            d.wait()
        pltpu.sync_copy(rows_v, out_hbm.at[pl.ds(base, _CHUNK)])
        return carry

    lax.fori_loop(0, _NCHUNK, chunk_body, 0)


@jax.jit
def kernel(indices, embed_table):
    idx_flat = indices.reshape(_TOTAL).astype(jnp.int32)
    mesh = plsc.VectorSubcoreMesh(core_axis_name="c", subcore_axis_name="s")
    out = pl.kernel(
        _embed_gather,
        mesh=mesh,
        out_type=jax.ShapeDtypeStruct((_TOTAL, EMBED_DIM), jnp.float32),
        scratch_types=[
            pltpu.VMEM((_CHUNK,), jnp.int32),
            pltpu.VMEM((_CHUNK, EMBED_DIM), jnp.float32),
            pltpu.SemaphoreType.DMA,
        ],
        compiler_params=pltpu.CompilerParams(use_tc_tiling_on_sc=False),
    )(idx_flat, embed_table)
    return out.reshape(BATCH, SEQ, EMBED_DIM)


# speedup vs baseline: 1.0277x; 1.0277x over previous
import jax
import jax.numpy as jnp
from jax import lax
from jax.experimental import pallas as pl
from jax.experimental.pallas import tpu as pltpu
from jax.experimental.pallas import tpu_sc as plsc

VOCAB = 1000000
EMBED_DIM = 64
BATCH = 4096
SEQ = 200

_NC = 2
_NS = 16
_NW = _NC * _NS

_TOTAL = BATCH * SEQ
_PER_W = _TOTAL // _NW
_GATHER = 128
_K = 5
_CHUNK = _K * _GATHER
_NCHUNK = _PER_W // _CHUNK
_NBUF = 2


def _embed_gather(idx_hbm, table_hbm, out_hbm,
                  idx0, idx1, rows0, rows1, sg0, sg1, sw0, sw1):
    wid = lax.axis_index("s") * _NC + lax.axis_index("c")
    base_w = wid * _PER_W
    idx_v = (idx0, idx1)
    rows_v = (rows0, rows1)
    sem_g = (sg0, sg1)
    sem_w = (sw0, sw1)

    def fire_gathers(b, chunk):
        base = base_w + chunk * _CHUNK
        pltpu.sync_copy(idx_hbm.at[pl.ds(base, _CHUNK)], idx_v[b])
        for j in range(_K):
            pltpu.async_copy(
                table_hbm.at[idx_v[b].at[pl.ds(j * _GATHER, _GATHER)]],
                rows_v[b].at[pl.ds(j * _GATHER, _GATHER)],
                sem_g[b],
            )

    def drain_gathers(b):
        for j in range(_K):
            pltpu.make_async_copy(
                table_hbm.at[idx_v[b].at[pl.ds(j * _GATHER, _GATHER)]],
                rows_v[b].at[pl.ds(j * _GATHER, _GATHER)],
                sem_g[b],
            ).wait()

    def fire_writeback(b, chunk):
        base = base_w + chunk * _CHUNK
        pltpu.async_copy(rows_v[b], out_hbm.at[pl.ds(base, _CHUNK)], sem_w[b])

    def wait_writeback(b, chunk):
        base = base_w + chunk * _CHUNK
        pltpu.make_async_copy(
            rows_v[b], out_hbm.at[pl.ds(base, _CHUNK)], sem_w[b]
        ).wait()

    for b in range(_NBUF):
        fire_gathers(b, b)

    def step(s, carry):
        for b in range(_NBUF):
            chunk = _NBUF * s + b
            drain_gathers(b)
            fire_writeback(b, chunk)
            wait_writeback(b, chunk)
            fire_gathers(b, chunk + _NBUF)
        return carry

    lax.fori_loop(0, _NCHUNK // _NBUF - 1, step, 0)

    for b in range(_NBUF):
        chunk = _NCHUNK - _NBUF + b
        drain_gathers(b)
        fire_writeback(b, chunk)
        wait_writeback(b, chunk)


@jax.jit
def kernel(indices, embed_table):
    idx_flat = indices.reshape(_TOTAL).astype(jnp.int32)
    mesh = plsc.VectorSubcoreMesh(core_axis_name="c", subcore_axis_name="s")
    out = pl.kernel(
        _embed_gather,
        mesh=mesh,
        out_type=jax.ShapeDtypeStruct((_TOTAL, EMBED_DIM), jnp.float32),
        scratch_types=[
            pltpu.VMEM((_CHUNK,), jnp.int32),
            pltpu.VMEM((_CHUNK,), jnp.int32),
            pltpu.VMEM((_CHUNK, EMBED_DIM), jnp.float32),
            pltpu.VMEM((_CHUNK, EMBED_DIM), jnp.float32),
            pltpu.SemaphoreType.DMA,
            pltpu.SemaphoreType.DMA,
            pltpu.SemaphoreType.DMA,
            pltpu.SemaphoreType.DMA,
        ],
        compiler_params=pltpu.CompilerParams(use_tc_tiling_on_sc=False),
    )(idx_flat, embed_table)
    return out.reshape(BATCH, SEQ, EMBED_DIM)


# speedup vs baseline: 1.0344x; 1.0065x over previous
import jax
import jax.numpy as jnp
from jax import lax
from jax.experimental import pallas as pl
from jax.experimental.pallas import tpu as pltpu
from jax.experimental.pallas import tpu_sc as plsc

VOCAB = 1000000
EMBED_DIM = 64
BATCH = 4096
SEQ = 200

_NC = 2
_NS = 16
_NW = _NC * _NS

_TOTAL = BATCH * SEQ
_PER_W = _TOTAL // _NW
_CHUNK = 640
_NCHUNK = _PER_W // _CHUNK
_NBUF = 2


def _embed_gather(idx_hbm, table_hbm, out_hbm,
                  idx_all, rows0, rows1, sg0, sg1, sw0, sw1):
    wid = lax.axis_index("s") * _NC + lax.axis_index("c")
    base_w = wid * _PER_W
    rows_v = (rows0, rows1)
    sem_g = (sg0, sg1)
    sem_w = (sw0, sw1)

    pltpu.sync_copy(idx_hbm.at[pl.ds(base_w, _PER_W)], idx_all)

    def idx_slice(chunk):
        return idx_all.at[pl.ds(chunk * _CHUNK, _CHUNK)]

    def fire_gather(b, chunk):
        pltpu.async_copy(table_hbm.at[idx_slice(chunk)], rows_v[b], sem_g[b])

    def drain_gather(b, chunk):
        pltpu.make_async_copy(
            table_hbm.at[idx_slice(chunk)], rows_v[b], sem_g[b]).wait()

    def fire_writeback(b, chunk):
        base = base_w + chunk * _CHUNK
        pltpu.async_copy(rows_v[b], out_hbm.at[pl.ds(base, _CHUNK)], sem_w[b])

    def wait_writeback(b, chunk):
        base = base_w + chunk * _CHUNK
        pltpu.make_async_copy(
            rows_v[b], out_hbm.at[pl.ds(base, _CHUNK)], sem_w[b]).wait()

    for b in range(_NBUF):
        fire_gather(b, b)

    def step(s, carry):
        for b in range(_NBUF):
            chunk = _NBUF * s + b
            drain_gather(b, chunk)
            fire_writeback(b, chunk)
            wait_writeback(b, chunk)
            fire_gather(b, chunk + _NBUF)
        return carry

    lax.fori_loop(0, _NCHUNK // _NBUF - 1, step, 0)

    for b in range(_NBUF):
        chunk = _NCHUNK - _NBUF + b
        drain_gather(b, chunk)
        fire_writeback(b, chunk)
        wait_writeback(b, chunk)


@jax.jit
def kernel(indices, embed_table):
    idx_flat = indices.reshape(_TOTAL).astype(jnp.int32)
    mesh = plsc.VectorSubcoreMesh(core_axis_name="c", subcore_axis_name="s")
    out = pl.kernel(
        _embed_gather,
        mesh=mesh,
        out_type=jax.ShapeDtypeStruct((_TOTAL, EMBED_DIM), jnp.float32),
        scratch_types=[
            pltpu.VMEM((_PER_W,), jnp.int32),
            pltpu.VMEM((_CHUNK, EMBED_DIM), jnp.float32),
            pltpu.VMEM((_CHUNK, EMBED_DIM), jnp.float32),
            pltpu.SemaphoreType.DMA,
            pltpu.SemaphoreType.DMA,
            pltpu.SemaphoreType.DMA,
            pltpu.SemaphoreType.DMA,
        ],
        compiler_params=pltpu.CompilerParams(use_tc_tiling_on_sc=False),
    )(idx_flat, embed_table)
    return out.reshape(BATCH, SEQ, EMBED_DIM)


# speedup vs baseline: 1.0613x; 1.0260x over previous
import jax
import jax.numpy as jnp
from jax import lax
from jax.experimental import pallas as pl
from jax.experimental.pallas import tpu as pltpu
from jax.experimental.pallas import tpu_sc as plsc

VOCAB = 1000000
EMBED_DIM = 64
BATCH = 4096
SEQ = 200

_NC = 2
_NS = 16
_NW = _NC * _NS

_TOTAL = BATCH * SEQ
_PER_W = _TOTAL // _NW
_CHUNK = 640
_NCHUNK = _PER_W // _CHUNK
_NBUF = 2


def _embed_gather(idx_hbm, table_hbm, out_hbm,
                  idx_all, rows0, rows1, sg0, sg1, sw0, sw1):
    wid = lax.axis_index("s") * _NC + lax.axis_index("c")
    base_w = wid * _PER_W
    rows_v = (rows0, rows1)
    sem_g = (sg0, sg1)
    sem_w = (sw0, sw1)

    pltpu.sync_copy(idx_hbm.at[pl.ds(base_w, _PER_W)], idx_all)

    def idx_slice(chunk):
        return idx_all.at[pl.ds(chunk * _CHUNK, _CHUNK)]

    def fire_gather(b, chunk):
        pltpu.async_copy(table_hbm.at[idx_slice(chunk)], rows_v[b], sem_g[b])

    def drain_gather(b, chunk):
        pltpu.make_async_copy(
            table_hbm.at[idx_slice(chunk)], rows_v[b], sem_g[b]).wait()

    def fire_writeback(b, chunk):
        base = base_w + chunk * _CHUNK
        pltpu.async_copy(rows_v[b], out_hbm.at[pl.ds(base, _CHUNK)], sem_w[b])

    def wait_writeback(b, chunk):
        base = base_w + chunk * _CHUNK
        pltpu.make_async_copy(
            rows_v[b], out_hbm.at[pl.ds(base, _CHUNK)], sem_w[b]).wait()

    for b in range(_NBUF):
        fire_gather(b, b)

    def step(s, carry):
        for b in range(_NBUF):
            chunk = _NBUF * s + b
            drain_gather(b, chunk)
            fire_writeback(b, chunk)
            wait_writeback(b, chunk)
            fire_gather(b, chunk + _NBUF)
        return carry

    lax.fori_loop(0, _NCHUNK // _NBUF - 1, step, 0)

    for b in range(_NBUF):
        chunk = _NCHUNK - _NBUF + b
        drain_gather(b, chunk)
        fire_writeback(b, chunk)
        wait_writeback(b, chunk)


@jax.jit
def kernel(indices, embed_table):
    idx_flat = indices.T.reshape(_TOTAL).astype(jnp.int32)
    mesh = plsc.VectorSubcoreMesh(core_axis_name="c", subcore_axis_name="s")
    out = pl.kernel(
        _embed_gather,
        mesh=mesh,
        out_type=jax.ShapeDtypeStruct((_TOTAL, EMBED_DIM), jnp.float32),
        scratch_types=[
            pltpu.VMEM((_PER_W,), jnp.int32),
            pltpu.VMEM((_CHUNK, EMBED_DIM), jnp.float32),
            pltpu.VMEM((_CHUNK, EMBED_DIM), jnp.float32),
            pltpu.SemaphoreType.DMA,
            pltpu.SemaphoreType.DMA,
            pltpu.SemaphoreType.DMA,
            pltpu.SemaphoreType.DMA,
        ],
        compiler_params=pltpu.CompilerParams(use_tc_tiling_on_sc=False),
    )(idx_flat, embed_table)
    return out.reshape(SEQ, BATCH, EMBED_DIM).transpose(1, 0, 2)
